# baseline (speedup 1.0000x reference)
import functools

import jax
import jax.numpy as jnp
from jax.experimental import pallas as pl

TOPK = 10
BETA = 0.25
GTOP = 4
GLANES = 128


def _l2n(x):
    n = jnp.sqrt(jnp.sum(x * x, axis=1, keepdims=True))
    return x / jnp.clip(n, 1e-12)


def _norm_books_kernel(a_ref, b_ref, c_ref, d_ref,
                       an_ref, bn_ref, cn_ref, dn_ref,
                       aq_ref, bq_ref, cq_ref, dq_ref):
    for r, o, q in ((a_ref, an_ref, aq_ref), (b_ref, bn_ref, bq_ref),
                    (c_ref, cn_ref, cq_ref), (d_ref, dn_ref, dq_ref)):
        en = _l2n(r[...])
        o[...] = en
        q[...] = jnp.sum(en * en, axis=1, keepdims=True).T


def _topk_weights(d, tb, n_e):
    ngrp = n_e // GLANES
    dm = d.reshape(tb, ngrp, GLANES)
    tops, cnts = [], []
    for k in range(GTOP):
        gm = jnp.min(dm, axis=2)
        eq = dm == gm[:, :, None]
        tops.append(gm)
        cnts.append(jnp.sum(eq.astype(jnp.float32), axis=2))
        if k < GTOP - 1:
            dm = jnp.where(eq, jnp.inf, dm)
    tt = jnp.concatenate(tops, axis=1)
    tc = jnp.concatenate(cnts, axis=1)
    dmin = None
    t = None
    cum = jnp.zeros((tb, 1), jnp.float32)
    for k in range(TOPK):
        mv = jnp.min(tt, axis=1, keepdims=True)
        if k == 0:
            dmin = mv
            t = mv
        else:
            t = jnp.where(cum < TOPK, mv, t)
        eq = tt == mv
        cum = cum + jnp.sum(jnp.where(eq, tc, 0.0), axis=1, keepdims=True)
        if k < TOPK - 1:
            tt = jnp.where(eq, jnp.inf, tt)
    p = jnp.where(d <= t, jnp.exp(dmin - d), 0.0)
    z = jnp.sum(p, axis=1, keepdims=True)
    pres = (jnp.max(p, axis=0, keepdims=True) > 0.0).astype(jnp.float32)
    return p, z, pres


def _mmt(a, b_t):
    return jax.lax.dot_general(a, b_t, (((1,), (1,)), ((), ())),
                               preferred_element_type=jnp.float32)


def _dist(xn, en, nsq):
    sx2 = jnp.sum(xn * xn, axis=1, keepdims=True)
    return (sx2 + nsq) - 2.0 * _mmt(xn, en)


def _shared_kernel(z_ref, est_ref, esg_ref, nst_ref, nsg_ref,
                   zq_ref, ss_ref, pres_ref, *, tb, n_e, d_half):
    i = pl.program_id(0)

    @pl.when(i == 0)
    def _init():
        ss_ref[...] = jnp.zeros_like(ss_ref)
        pres_ref[...] = jnp.zeros_like(pres_ref)

    zb = z_ref[...]
    zt = zb[:, :d_half]
    zg = zb[:, d_half:]
    est = est_ref[...]
    esg = esg_ref[...]
    d = (_dist(_l2n(zt), est, nst_ref[...])
         + _dist(_l2n(zg), esg, nsg_ref[...]))
    p, zden, pres = _topk_weights(d, tb, n_e)
    zq_l = jnp.dot(p, est, preferred_element_type=jnp.float32)
    zq_r = jnp.dot(p, esg, preferred_element_type=jnp.float32)
    zq = jnp.concatenate([zq_l, zq_r], axis=1) / zden
    zq_ref[...] = zb + (zq - zb)
    ss_ref[...] += jnp.sum((zq - zb) ** 2).reshape(1, 1)
    pres_ref[...] = jnp.maximum(pres_ref[...], pres)


def _specific_kernel(z_ref, eb_ref, nsq_ref, zq_ref, ss_ref, pres_ref,
                     *, tb, n_e, d_half, half):
    i = pl.program_id(0)

    @pl.when(i == 0)
    def _init():
        ss_ref[...] = jnp.zeros_like(ss_ref)
        pres_ref[...] = jnp.zeros_like(pres_ref)

    zb = z_ref[...]
    zh = zb[:, :d_half] if half == 0 else zb[:, d_half:]
    eb = eb_ref[...]
    d = _dist(_l2n(zh), eb, nsq_ref[...])
    p, zden, pres = _topk_weights(d, tb, n_e)
    zq = jnp.dot(p, eb, preferred_element_type=jnp.float32) / zden
    zq_ref[...] = zh + (zq - zh)
    ss_ref[...] += jnp.sum((zq - zh) ** 2).reshape(1, 1)
    pres_ref[...] = jnp.maximum(pres_ref[...], pres)


def kernel(z, W_shared_text, W_shared_graph, W_text, W_graph):
    n, d = z.shape
    n_e, d_half = W_text.shape
    f32 = jnp.float32

    nb_blk = min(1024, n_e)
    outs = pl.pallas_call(
        _norm_books_kernel,
        grid=(n_e // nb_blk,),
        in_specs=[pl.BlockSpec((nb_blk, d_half), lambda i: (i, 0))] * 4,
        out_specs=([pl.BlockSpec((nb_blk, d_half), lambda i: (i, 0))] * 4
                   + [pl.BlockSpec((1, nb_blk), lambda i: (0, i))] * 4),
        out_shape=([jax.ShapeDtypeStruct((n_e, d_half), f32)] * 4
                   + [jax.ShapeDtypeStruct((1, n_e), f32)] * 4),
    )(W_shared_text, W_shared_graph, W_text, W_graph)
    estn, esgn, etn, egn, nst, nsg, nt, ng = outs

    tb = 192 if n % 192 == 0 else (128 if n % 128 == 0 else n)
    grid = (n // tb,)
    z_spec = pl.BlockSpec((tb, d), lambda i: (i, 0))
    book_spec = pl.BlockSpec((n_e, d_half), lambda i: (0, 0))
    nsq_spec = pl.BlockSpec((1, n_e), lambda i: (0, 0))
    acc_spec = pl.BlockSpec((1, 1), lambda i: (0, 0))
    pres_spec = pl.BlockSpec((1, n_e), lambda i: (0, 0))

    zq_sh, ss_sh, pres_sh = pl.pallas_call(
        functools.partial(_shared_kernel, tb=tb, n_e=n_e, d_half=d_half),
        grid=grid,
        in_specs=[z_spec, book_spec, book_spec, nsq_spec, nsq_spec],
        out_specs=[pl.BlockSpec((tb, d), lambda i: (i, 0)), acc_spec, pres_spec],
        out_shape=[jax.ShapeDtypeStruct((n, d), f32),
                   jax.ShapeDtypeStruct((1, 1), f32),
                   jax.ShapeDtypeStruct((1, n_e), f32)],
    )(z, estn, esgn, nst, nsg)

    def specific(book, nsq, half):
        return pl.pallas_call(
            functools.partial(_specific_kernel, tb=tb, n_e=n_e,
                              d_half=d_half, half=half),
            grid=grid,
            in_specs=[z_spec, book_spec, nsq_spec],
            out_specs=[pl.BlockSpec((tb, d_half), lambda i: (i, 0)),
                       acc_spec, pres_spec],
            out_shape=[jax.ShapeDtypeStruct((n, d_half), f32),
                       jax.ShapeDtypeStruct((1, 1), f32),
                       jax.ShapeDtypeStruct((1, n_e), f32)],
        )(z, book, nsq)

    zq_t, ss_t, pres_t = specific(etn, nt, 0)
    zq_g, ss_g, pres_g = specific(egn, ng, 1)

    zt = z[:, :d_half]
    zg = z[:, d_half:]
    vq_sh = ss_sh[0, 0] / (n * d)
    vq_t = ss_t[0, 0] / (n * d_half)
    vq_g = ss_g[0, 0] / (n * d_half)

    def usage(pres):
        return (jnp.sum(pres) + (1.0 - pres[0, 0])) / n_e

    return (zq_sh, zq_t, zq_g, zt, zg,
            vq_sh, BETA * vq_sh, vq_t, BETA * vq_t, vq_g, BETA * vq_g,
            usage(pres_sh), usage(pres_t), usage(pres_g))


# speedup vs baseline: 1.7213x; 1.7213x over previous
import functools

import jax
import jax.numpy as jnp
from jax.experimental import pallas as pl

TOPK = 10
BETA = 0.25


def _l2n(x):
    n = jnp.sqrt(jnp.sum(x * x, axis=1, keepdims=True))
    return x / jnp.clip(n, 1e-12)


def _norm_books_kernel(a_ref, b_ref, c_ref, d_ref,
                       an_ref, bn_ref, cn_ref, dn_ref,
                       aq_ref, bq_ref, cq_ref, dq_ref):
    for r, o, q in ((a_ref, an_ref, aq_ref), (b_ref, bn_ref, bq_ref),
                    (c_ref, cn_ref, cq_ref), (d_ref, dn_ref, dq_ref)):
        en = _l2n(r[...])
        o[...] = en
        q[...] = jnp.sum(en * en, axis=1, keepdims=True).T


def _topk_weights(d):
    dm = d
    dmin = None
    t = None
    for k in range(TOPK):
        mv = jnp.min(dm, axis=1, keepdims=True)
        if k == 0:
            dmin = mv
        t = mv
        if k < TOPK - 1:
            dm = jnp.where(dm == mv, jnp.inf, dm)
    p = jnp.where(d <= t, jnp.exp(dmin - d), 0.0)
    z = jnp.sum(p, axis=1, keepdims=True)
    pres = (jnp.max(p, axis=0, keepdims=True) > 0.0).astype(jnp.float32)
    return p, z, pres


def _mmt(a, b_t):
    return jax.lax.dot_general(a, b_t, (((1,), (1,)), ((), ())),
                               preferred_element_type=jnp.float32)


def _dist(xn, en, nsq):
    sx2 = jnp.sum(xn * xn, axis=1, keepdims=True)
    return (sx2 + nsq) - 2.0 * _mmt(xn, en)


def _shared_kernel(z_ref, est_ref, esg_ref, nst_ref, nsg_ref,
                   zq_ref, ss_ref, pres_ref, *, d_half):
    i = pl.program_id(0)

    @pl.when(i == 0)
    def _init():
        ss_ref[...] = jnp.zeros_like(ss_ref)
        pres_ref[...] = jnp.zeros_like(pres_ref)

    zb = z_ref[...]
    zt = zb[:, :d_half]
    zg = zb[:, d_half:]
    est = est_ref[...]
    esg = esg_ref[...]
    d = (_dist(_l2n(zt), est, nst_ref[...])
         + _dist(_l2n(zg), esg, nsg_ref[...]))
    p, zden, pres = _topk_weights(d)
    zq_l = jnp.dot(p, est, preferred_element_type=jnp.float32)
    zq_r = jnp.dot(p, esg, preferred_element_type=jnp.float32)
    zq = jnp.concatenate([zq_l, zq_r], axis=1) / zden
    zq_ref[...] = zb + (zq - zb)
    ss_ref[...] += jnp.sum((zq - zb) ** 2).reshape(1, 1)
    pres_ref[...] = jnp.maximum(pres_ref[...], pres)


def _specific_kernel(z_ref, eb_ref, nsq_ref, zq_ref, ss_ref, pres_ref,
                     *, d_half, half):
    i = pl.program_id(0)

    @pl.when(i == 0)
    def _init():
        ss_ref[...] = jnp.zeros_like(ss_ref)
        pres_ref[...] = jnp.zeros_like(pres_ref)

    zb = z_ref[...]
    zh = zb[:, :d_half] if half == 0 else zb[:, d_half:]
    eb = eb_ref[...]
    d = _dist(_l2n(zh), eb, nsq_ref[...])
    p, zden, pres = _topk_weights(d)
    zq = jnp.dot(p, eb, preferred_element_type=jnp.float32) / zden
    zq_ref[...] = zh + (zq - zh)
    ss_ref[...] += jnp.sum((zq - zh) ** 2).reshape(1, 1)
    pres_ref[...] = jnp.maximum(pres_ref[...], pres)


def kernel(z, W_shared_text, W_shared_graph, W_text, W_graph):
    n, d = z.shape
    n_e, d_half = W_text.shape
    f32 = jnp.float32

    nb_blk = min(1024, n_e)
    outs = pl.pallas_call(
        _norm_books_kernel,
        grid=(n_e // nb_blk,),
        in_specs=[pl.BlockSpec((nb_blk, d_half), lambda i: (i, 0))] * 4,
        out_specs=([pl.BlockSpec((nb_blk, d_half), lambda i: (i, 0))] * 4
                   + [pl.BlockSpec((1, nb_blk), lambda i: (0, i))] * 4),
        out_shape=([jax.ShapeDtypeStruct((n_e, d_half), f32)] * 4
                   + [jax.ShapeDtypeStruct((1, n_e), f32)] * 4),
    )(W_shared_text, W_shared_graph, W_text, W_graph)
    estn, esgn, etn, egn, nst, nsg, nt, ng = outs

    tb = min(256, n)
    grid = (n // tb,)
    z_spec = pl.BlockSpec((tb, d), lambda i: (i, 0))
    book_spec = pl.BlockSpec((n_e, d_half), lambda i: (0, 0))
    nsq_spec = pl.BlockSpec((1, n_e), lambda i: (0, 0))
    acc_spec = pl.BlockSpec((1, 1), lambda i: (0, 0))
    pres_spec = pl.BlockSpec((1, n_e), lambda i: (0, 0))

    zq_sh, ss_sh, pres_sh = pl.pallas_call(
        functools.partial(_shared_kernel, d_half=d_half),
        grid=grid,
        in_specs=[z_spec, book_spec, book_spec, nsq_spec, nsq_spec],
        out_specs=[pl.BlockSpec((tb, d), lambda i: (i, 0)), acc_spec, pres_spec],
        out_shape=[jax.ShapeDtypeStruct((n, d), f32),
                   jax.ShapeDtypeStruct((1, 1), f32),
                   jax.ShapeDtypeStruct((1, n_e), f32)],
    )(z, estn, esgn, nst, nsg)

    def specific(book, nsq, half):
        return pl.pallas_call(
            functools.partial(_specific_kernel, d_half=d_half, half=half),
            grid=grid,
            in_specs=[z_spec, book_spec, nsq_spec],
            out_specs=[pl.BlockSpec((tb, d_half), lambda i: (i, 0)),
                       acc_spec, pres_spec],
            out_shape=[jax.ShapeDtypeStruct((n, d_half), f32),
                       jax.ShapeDtypeStruct((1, 1), f32),
                       jax.ShapeDtypeStruct((1, n_e), f32)],
        )(z, book, nsq)

    zq_t, ss_t, pres_t = specific(etn, nt, 0)
    zq_g, ss_g, pres_g = specific(egn, ng, 1)

    zt = z[:, :d_half]
    zg = z[:, d_half:]
    vq_sh = ss_sh[0, 0] / (n * d)
    vq_t = ss_t[0, 0] / (n * d_half)
    vq_g = ss_g[0, 0] / (n * d_half)

    def usage(pres):
        return (jnp.sum(pres) + (1.0 - pres[0, 0])) / n_e

    return (zq_sh, zq_t, zq_g, zt, zg,
            vq_sh, BETA * vq_sh, vq_t, BETA * vq_t, vq_g, BETA * vq_g,
            usage(pres_sh), usage(pres_t), usage(pres_g))
